# baseline (speedup 1.0000x reference)
import functools

import jax
import jax.numpy as jnp
from jax import lax
from jax.experimental import pallas as pl
from jax.experimental.pallas import tpu as pltpu
from jax.experimental.pallas import tpu_sc as plsc

H = 128
HB = H // 16
KE = 32
NCORES = 2
NSUB = 16
NWORK = NCORES * NSUB
SPMEM_WORDS = 2_000_000


def _round_up(x, m):
    return (x + m - 1) // m * m


def _pick_rounds(n_dst, epw):
    fixed = 5 * epw + 16 * NSUB + 2 * KE + 4 * KE * H + 2 * H + 16 * KE + 128
    rounds = 1
    while True:
        rng = _round_up(-(-n_dst // rounds), NSUB * 8)
        if NSUB * (fixed + rng) + H * rng <= SPMEM_WORDS:
            return rounds, rng
        rounds += 1


def _lin_table_body(x_ref, w_ref, b_ref, out_ref, *, n, br):
    i = pl.program_id(0)
    rows = i * br + lax.broadcasted_iota(jnp.int32, (br, 1), 0)
    valid = rows < n
    y = jnp.dot(x_ref[...], w_ref[...], preferred_element_type=jnp.float32)
    y = y + b_ref[...]
    out_ref[...] = jnp.where(valid, y, 0.0)


def _lin_table(x, w, b, n):
    br = 256
    npad = n + 1
    return pl.pallas_call(
        functools.partial(_lin_table_body, n=n, br=br),
        grid=(pl.cdiv(npad, br),),
        in_specs=[
            pl.BlockSpec((br, H), lambda i: (i, 0)),
            pl.BlockSpec((H, H), lambda i: (0, 0)),
            pl.BlockSpec((1, H), lambda i: (0, 0)),
        ],
        out_specs=pl.BlockSpec((br, H), lambda i: (i, 0)),
        out_shape=jax.ShapeDtypeStruct((npad, H), jnp.float32),
    )(x, w, b)


def _update_body(p0_ref, p1_ref, c_ref, old_ref, wu_ref, bu_ref, g_ref,
                 bl_ref, *rest, n, br, emit):
    if emit:
        wl_ref, blin_ref, new_ref, tbl_ref = rest
    else:
        (new_ref,) = rest
    i = pl.program_id(0)
    rows = i * br + lax.broadcasted_iota(jnp.int32, (br, 1), 0)
    valid = rows < n
    sums = p0_ref[...] + p1_ref[...]
    cnt = jnp.sum(c_ref[...], axis=0)[:, None]
    agg = sums / jnp.maximum(cnt, 1.0)
    x2 = (jnp.dot(agg, wu_ref[:H, :], preferred_element_type=jnp.float32)
          + jnp.dot(old_ref[...], wu_ref[H:, :],
                    preferred_element_type=jnp.float32)
          + bu_ref[...])
    mean = jnp.mean(x2, axis=-1, keepdims=True)
    var = jnp.mean((x2 - mean) ** 2, axis=-1, keepdims=True)
    y = (x2 - mean) * lax.rsqrt(var + 1e-5) * g_ref[...] + bl_ref[...]
    y = jnp.maximum(y, 0.0)
    y = jnp.where(valid, y, 0.0)
    new_ref[...] = y
    if emit:
        z = (jnp.dot(y, wl_ref[...], preferred_element_type=jnp.float32)
             + blin_ref[...])
        tbl_ref[...] = jnp.where(valid, z, 0.0)


def _update(p0, p1, cnts, old, wu, bu, g, bln, n, wl=None, bl=None):
    br = 256
    emit = wl is not None
    ngrid = n + 1 if emit else n
    in_specs = [
        pl.BlockSpec((br, H), lambda i: (i, 0)),
        pl.BlockSpec((br, H), lambda i: (i, 0)),
        pl.BlockSpec((NWORK, br), lambda i: (0, i)),
        pl.BlockSpec((br, H), lambda i: (i, 0)),
        pl.BlockSpec((2 * H, H), lambda i: (0, 0)),
        pl.BlockSpec((1, H), lambda i: (0, 0)),
        pl.BlockSpec((1, H), lambda i: (0, 0)),
        pl.BlockSpec((1, H), lambda i: (0, 0)),
    ]
    args = [p0, p1, cnts, old, wu, bu, g, bln]
    out_shape = [jax.ShapeDtypeStruct((n, H), jnp.float32)]
    out_specs = [pl.BlockSpec((br, H), lambda i: (i, 0))]
    if emit:
        in_specs += [pl.BlockSpec((H, H), lambda i: (0, 0)),
                     pl.BlockSpec((1, H), lambda i: (0, 0))]
        args += [wl, bl]
        out_shape.append(jax.ShapeDtypeStruct((n + 1, H), jnp.float32))
        out_specs.append(pl.BlockSpec((br, H), lambda i: (i, 0)))
    res = pl.pallas_call(
        functools.partial(_update_body, n=n, br=br, emit=emit),
        grid=(pl.cdiv(ngrid, br),),
        in_specs=in_specs,
        out_specs=out_specs,
        out_shape=out_shape,
    )(*args)
    return res


def _edge_pass_body(table, src2d, dst2d, ew2d, gw, gb, zeros, zeros1,
                    out, outc,
                    shared, srcv, dstv, ewv, posb, posb2, startb, endb,
                    rows, msg, rows2, msg2, cntb,
                    gwv, gbv, ewtmp, mtmp, svtmp, dltmp,
                    ewtmp2, mtmp2, svtmp2, dltmp2,
                    *, epw, n_dst, n_src, rounds, rng, rpt):
    c = lax.axis_index("c")
    s = lax.axis_index("s")
    w = c * NSUB + s
    pltpu.sync_copy(src2d.at[w], srcv)
    pltpu.sync_copy(dst2d.at[w], dstv)
    pltpu.sync_copy(ew2d.at[w], ewv)
    pltpu.sync_copy(gw, gwv)
    pltpu.sync_copy(gb, gbv)
    gwr = [gwv[pl.ds(16 * b, 16)] for b in range(HB)]
    gbr = [gbv[pl.ds(16 * b, 16)] for b in range(HB)]
    iota = lax.iota(jnp.int32, 16)
    ones = jnp.full((16,), 1.0, jnp.float32)
    nchunks_raw = epw // 16

    lane0m = iota == jnp.full((16,), 0, jnp.int32)

    def zb(i, cz):
        posb[pl.ds(16 * i, 16)] = jnp.zeros((16,), jnp.int32)
        return cz

    lax.fori_loop(0, (epw + KE) // 16, zb, jnp.int32(0))

    def zb2(i, cz):
        posb2[pl.ds(16 * i, 16)] = jnp.zeros((16,), jnp.int32)
        return cz

    lax.fori_loop(0, (epw + 16 * NSUB + KE) // 16, zb2, jnp.int32(0))

    def round_body(r, carry):
        lo = r * rng
        hi = jnp.minimum(lo + rng, n_dst)
        pltpu.sync_copy(zeros.at[pl.ds(s * rpt, rpt)],
                        shared.at[pl.ds(s * rpt, rpt)])
        pltpu.sync_copy(zeros1, cntb)
        plsc.subcore_barrier()

        def cbody(i, nn):
            d = dstv[pl.ds(16 * i, 16)]
            m = (d >= jnp.full((16,), lo, jnp.int32)) & (
                d < jnp.full((16,), hi, jnp.int32))
            cum = plsc.cumsum(m.astype(jnp.int32))
            idx = jnp.full((16,), nn - 1, jnp.int32) + cum
            pos = iota + jnp.full((16,), 16 * i, jnp.int32)
            plsc.store_scatter(posb, [idx], pos, mask=m)
            return nn + jnp.max(cum)

        n = lax.fori_loop(0, nchunks_raw, cbody, jnp.int32(0))

        def qbody(q, cur):
            qlo = lo + q * rpt
            qhi = jnp.minimum(qlo + rpt, hi)

            def c2(i, nn):
                pos = posb[pl.ds(16 * i, 16)]
                dv = plsc.load_gather(dstv, [pos])
                lanev = iota + jnp.full((16,), 16 * i, jnp.int32)
                m = ((lanev < jnp.full((16,), n, jnp.int32))
                     & (dv >= jnp.full((16,), qlo, jnp.int32))
                     & (dv < jnp.full((16,), qhi, jnp.int32)))
                cum = plsc.cumsum(m.astype(jnp.int32))
                idx = jnp.full((16,), nn - 1, jnp.int32) + cum
                plsc.store_scatter(posb2, [idx], pos, mask=m)
                return nn + jnp.max(cum)

            nq = lax.fori_loop(0, (n + 15) // 16, c2, cur)
            qsel = jnp.full((16,), q, jnp.int32)
            plsc.store_scatter(startb, [qsel],
                               jnp.full((16,), cur, jnp.int32), mask=lane0m)
            plsc.store_scatter(endb, [qsel],
                               jnp.full((16,), nq, jnp.int32), mask=lane0m)
            return (nq + 15) // 16 * 16

        lax.fori_loop(0, NSUB, qbody, jnp.int32(0))

        def sstep(p, cc):
            q = jnp.remainder(s + p, NSUB)
            qsel = jnp.full((16,), q, jnp.int32)
            st = jnp.max(plsc.load_gather(startb, [qsel]))
            ln = jnp.max(plsc.load_gather(endb, [qsel])) - st

            def chunk(j, rowsb, msgb, ewtmpb, mtmpb, svtmpb, dltmpb):
                evs = []
                mvs = []
                for k in range(KE // 16):
                    pos = posb2[pl.ds(st + KE * j + 16 * k, 16)]
                    sv = plsc.load_gather(srcv, [pos])
                    dv = plsc.load_gather(dstv, [pos])
                    ev = plsc.load_gather(ewv, [pos])
                    m = (iota + jnp.full((16,), KE * j + 16 * k, jnp.int32)
                         ) < jnp.full((16,), ln, jnp.int32)
                    sv = jnp.where(m, sv, jnp.full((16,), n_src, jnp.int32))
                    dl = jnp.clip(
                        jnp.where(m, dv, jnp.full((16,), lo, jnp.int32))
                        - jnp.full((16,), lo, jnp.int32),
                        jnp.full((16,), 0, jnp.int32),
                        jnp.full((16,), rng - 1, jnp.int32))
                    evs.append(ev)
                    mvs.append(m.astype(jnp.float32))
                    svtmpb[pl.ds(16 * k, 16)] = sv
                    dltmpb[0, pl.ds(16 * k, 16)] = dl
                    for e in range(16):
                        lane = iota == jnp.full((16,), e, jnp.int32)
                        plsc.addupdate_scatter(cntb, [dl], ones,
                                               mask=m & lane)
                pltpu.sync_copy(table.at[svtmpb.at[:]], rowsb)
                for e in range(KE):
                    lane = iota == jnp.full((16,), e % 16, jnp.int32)
                    ewb = jnp.full(
                        (16,), jnp.sum(jnp.where(lane, evs[e // 16], 0.0)),
                        jnp.float32)
                    mv = jnp.full(
                        (16,), jnp.sum(jnp.where(lane, mvs[e // 16], 0.0)),
                        jnp.float32)
                    for b in range(HB):
                        rowb = rowsb[e, pl.ds(16 * b, 16)]
                        t = ewb * gwr[b] + gbr[b]
                        sg = mv / (1.0 + jnp.exp(-t))
                        msgb[e, pl.ds(16 * b, 16)] = rowb * sg
                pltpu.sync_copy(msgb, shared.at[dltmpb.at[0]], add=True)

            def pbody(j, carry2):
                @pl.when(jnp.remainder(j, 2) == 0)
                def _():
                    chunk(j, rows, msg, ewtmp, mtmp, svtmp, dltmp)

                @pl.when(jnp.remainder(j, 2) == 1)
                def _():
                    chunk(j, rows2, msg2, ewtmp2, mtmp2, svtmp2, dltmp2)

                return carry2

            lax.fori_loop(0, (ln + KE - 1) // KE, pbody, jnp.int32(0))
            plsc.subcore_barrier()
            return cc

        lax.fori_loop(0, NSUB, sstep, jnp.int32(0))
        pltpu.sync_copy(shared.at[pl.ds(s * rpt, rpt)],
                        out.at[c, pl.ds(lo + s * rpt, rpt)])
        pltpu.sync_copy(cntb, outc.at[w, pl.ds(lo, rng)])
        plsc.subcore_barrier()
        return carry

    lax.fori_loop(0, rounds, round_body, jnp.int32(0))


def _edge_pass(table, src2d, dst2d, ew2d, gw, gb, n_dst, n_src):
    epw = src2d.shape[1]
    rounds, rng = _pick_rounds(n_dst, epw)
    rpt = rng // NSUB
    n_out = rounds * rng
    zeros = jnp.zeros((rng, H), jnp.float32)
    zeros1 = jnp.zeros((rng,), jnp.float32)
    mesh = plsc.VectorSubcoreMesh(core_axis_name="c", subcore_axis_name="s")
    body = functools.partial(
        _edge_pass_body, epw=epw, n_dst=n_dst, n_src=n_src,
        rounds=rounds, rng=rng, rpt=rpt)
    f = pl.kernel(
        body,
        out_type=(
            jax.ShapeDtypeStruct((NCORES, n_out, H), jnp.float32),
            jax.ShapeDtypeStruct((NWORK, n_out), jnp.float32),
        ),
        mesh=mesh,
        compiler_params=pltpu.CompilerParams(needs_layout_passes=False),
        scratch_types=[
            pltpu.VMEM_SHARED((rng, H), jnp.float32),
            pltpu.VMEM((epw,), jnp.int32),
            pltpu.VMEM((epw,), jnp.int32),
            pltpu.VMEM((epw,), jnp.float32),
            pltpu.VMEM((epw + KE,), jnp.int32),
            pltpu.VMEM((epw + 16 * NSUB + KE,), jnp.int32),
            pltpu.VMEM((NSUB,), jnp.int32),
            pltpu.VMEM((NSUB,), jnp.int32),
            pltpu.VMEM((KE, H), jnp.float32),
            pltpu.VMEM((KE, H), jnp.float32),
            pltpu.VMEM((KE, H), jnp.float32),
            pltpu.VMEM((KE, H), jnp.float32),
            pltpu.VMEM((rng,), jnp.float32),
            pltpu.VMEM((H,), jnp.float32),
            pltpu.VMEM((H,), jnp.float32),
            pltpu.VMEM((KE,), jnp.float32),
            pltpu.VMEM((KE,), jnp.float32),
            pltpu.VMEM((KE,), jnp.int32),
            pltpu.VMEM((1, KE), jnp.int32),
            pltpu.VMEM((KE,), jnp.float32),
            pltpu.VMEM((KE,), jnp.float32),
            pltpu.VMEM((KE,), jnp.int32),
            pltpu.VMEM((1, KE), jnp.int32),
        ],
    )
    return f(table, src2d, dst2d, ew2d, gw, gb, zeros, zeros1)


def kernel(vh, ch, edge_index, ew,
           v2c_lin_w, v2c_lin_b, v2c_gate_w, v2c_gate_b,
           v2c_upd_w, v2c_upd_b, v2c_ln_g, v2c_ln_b,
           c2v_lin_w, c2v_lin_b, c2v_gate_w, c2v_gate_b,
           c2v_upd_w, c2v_upd_b, c2v_ln_g, c2v_ln_b):
    nv, h = vh.shape
    nc = ch.shape[0]
    e = ew.shape[0]
    assert h == H
    ci = edge_index[0]
    vi = edge_index[1]

    epw = _round_up(-(-e // NWORK), 16)
    epad = NWORK * epw
    vip = jnp.concatenate(
        [vi, jnp.full((epad - e,), nv, jnp.int32)]).reshape(NWORK, epw)
    cip = jnp.concatenate(
        [ci, jnp.full((epad - e,), nc, jnp.int32)]).reshape(NWORK, epw)
    ewp = jnp.concatenate(
        [ew, jnp.zeros((epad - e,), jnp.float32)]).reshape(NWORK, epw)

    r2 = lambda a: a.reshape(1, H)

    src1 = _lin_table(vh, v2c_lin_w, r2(v2c_lin_b), nv)
    part1, cnt1 = _edge_pass(src1, vip, cip, ewp, v2c_gate_w[0], v2c_gate_b,
                             n_dst=nc, n_src=nv)
    ch_new, src2 = _update(part1[0], part1[1], cnt1, ch, v2c_upd_w,
                           r2(v2c_upd_b), r2(v2c_ln_g), r2(v2c_ln_b), nc,
                           wl=c2v_lin_w, bl=r2(c2v_lin_b))
    part2, cnt2 = _edge_pass(src2, cip, vip, ewp, c2v_gate_w[0], c2v_gate_b,
                             n_dst=nv, n_src=nc)
    (vh_new,) = _update(part2[0], part2[1], cnt2, vh, c2v_upd_w,
                        r2(c2v_upd_b), r2(c2v_ln_g), r2(c2v_ln_b), nv)
    return vh_new, ch_new
